# baseline (speedup 1.0000x reference)
import jax
import jax.numpy as jnp
from jax import lax
from jax.experimental import pallas as pl
from jax.experimental.pallas import tpu as pltpu
from jax.experimental.pallas import tpu_sc as plsc

N = 10000
D = 128
NCLS = 64
NE = 320000
NP = 10240
NC = 2
NS = 16
NW = NC * NS
L = 16
EB = 128
NB = 80
NBH = NB // 2
EPW = NB * EB
EPAD = NW * EPW
RPW = NP // NS
HB = 64
HPS = NP // NC // NS
NZ = NP - N

_mesh = plsc.VectorSubcoreMesh(core_axis_name="c", subcore_axis_name="s",
                               num_cores=NC, num_subcores=NS)


def _deg_body(dst_hbm, zer_hbm, ones_hbm, out_hbm, dst_v, ones_v, acc_sh):
    c = lax.axis_index("c")
    s = lax.axis_index("s")
    wid = s * NC + c
    pltpu.sync_copy(dst_hbm.at[wid], dst_v)
    pltpu.sync_copy(ones_hbm, ones_v)
    pltpu.sync_copy(zer_hbm.at[pl.ds(s * RPW, RPW)],
                    acc_sh.at[pl.ds(s * RPW, RPW)])
    plsc.subcore_barrier()

    def step(j, carry):
        pltpu.sync_copy(ones_v, acc_sh.at[dst_v.at[j]], add=True)
        return carry

    lax.fori_loop(0, NB, step, 0)
    plsc.subcore_barrier()
    pltpu.sync_copy(acc_sh.at[pl.ds(s * RPW, RPW)],
                    out_hbm.at[c, pl.ds(s * RPW, RPW)])


_deg = pl.kernel(
    _deg_body,
    out_type=jax.ShapeDtypeStruct((NC, NP, D), jnp.float32),
    mesh=_mesh,
    scratch_types=[
        pltpu.VMEM((NB, EB), jnp.int32),
        pltpu.VMEM((EB, D), jnp.float32),
        pltpu.VMEM_SHARED((NP, D), jnp.float32),
    ],
)


def _agg_body(y_hbm, src_hbm, dst_hbm, zer_hbm, out_hbm,
              src_v, dst_v, rows_v, acc_sh, sem0, sem1, semS0, semS1):
    c = lax.axis_index("c")
    s = lax.axis_index("s")
    wid = s * NC + c
    pltpu.sync_copy(zer_hbm.at[pl.ds(s * RPW, RPW)],
                    acc_sh.at[pl.ds(s * RPW, RPW)])
    plsc.subcore_barrier()

    for ph in range(2):
        base = ph * NBH
        pltpu.sync_copy(src_hbm.at[wid, pl.ds(base, NBH)], src_v)
        pltpu.sync_copy(dst_hbm.at[wid, pl.ds(base, NBH)], dst_v)
        pltpu.async_copy(y_hbm.at[src_v.at[0]], rows_v.at[0], sem0)
        pltpu.async_copy(y_hbm.at[src_v.at[1]], rows_v.at[1], sem1)

        def step(t, carry):
            j0 = 2 * t
            pltpu.make_async_copy(y_hbm.at[src_v.at[j0]],
                                  rows_v.at[0], sem0).wait()
            pltpu.async_copy(rows_v.at[0], acc_sh.at[dst_v.at[j0]], semS0,
                             add=True)
            pltpu.make_async_copy(y_hbm.at[src_v.at[j0 + 1]],
                                  rows_v.at[1], sem1).wait()
            pltpu.async_copy(rows_v.at[1], acc_sh.at[dst_v.at[j0 + 1]], semS1,
                             add=True)
            pltpu.make_async_copy(rows_v.at[0], acc_sh.at[dst_v.at[j0]],
                                  semS0).wait()
            pltpu.async_copy(y_hbm.at[src_v.at[j0 + 2]], rows_v.at[0], sem0)
            pltpu.make_async_copy(rows_v.at[1], acc_sh.at[dst_v.at[j0 + 1]],
                                  semS1).wait()
            pltpu.async_copy(y_hbm.at[src_v.at[j0 + 3]], rows_v.at[1], sem1)
            return carry

        lax.fori_loop(0, NBH // 2 - 1, step, 0)
        pltpu.make_async_copy(y_hbm.at[src_v.at[NBH - 2]],
                              rows_v.at[0], sem0).wait()
        pltpu.sync_copy(rows_v.at[0], acc_sh.at[dst_v.at[NBH - 2]], add=True)
        pltpu.make_async_copy(y_hbm.at[src_v.at[NBH - 1]],
                              rows_v.at[1], sem1).wait()
        pltpu.sync_copy(rows_v.at[1], acc_sh.at[dst_v.at[NBH - 1]], add=True)

    plsc.subcore_barrier()
    pltpu.sync_copy(acc_sh.at[pl.ds(s * RPW, RPW)],
                    out_hbm.at[c, pl.ds(s * RPW, RPW)])


_agg = pl.kernel(
    _agg_body,
    out_type=jax.ShapeDtypeStruct((NC, NP, D), jnp.float32),
    mesh=_mesh,
    scratch_types=[
        pltpu.VMEM((NBH, EB), jnp.int32),
        pltpu.VMEM((NBH, EB), jnp.int32),
        pltpu.VMEM((2, EB, D), jnp.float32),
        pltpu.VMEM_SHARED((NP, D), jnp.float32),
        pltpu.SemaphoreType.DMA,
        pltpu.SemaphoreType.DMA,
        pltpu.SemaphoreType.DMA,
        pltpu.SemaphoreType.DMA,
    ],
)


BR = 1024
G = NP // BR


def _lin1_body(dp_ref, x_ref, w_ref, o_ref):
    dinv = lax.rsqrt(1.0 + dp_ref[0] + dp_ref[1])
    o_ref[...] = dinv * jnp.dot(x_ref[...], w_ref[...],
                                preferred_element_type=jnp.float32)


_lin1 = pl.pallas_call(
    _lin1_body,
    grid=(G,),
    in_specs=[
        pl.BlockSpec((NC, BR, D), lambda i: (0, i, 0)),
        pl.BlockSpec((BR, D), lambda i: (i, 0)),
        pl.BlockSpec((D, D), lambda i: (0, 0)),
    ],
    out_specs=pl.BlockSpec((BR, D), lambda i: (i, 0)),
    out_shape=jax.ShapeDtypeStruct((NP, D), jnp.float32),
)


def _mid_body(dp_ref, a_ref, y_ref, b_ref, w_ref, o_ref):
    dinv = lax.rsqrt(1.0 + dp_ref[0] + dp_ref[1])
    h = jnp.maximum(dinv * (a_ref[0] + a_ref[1] + y_ref[...]) + b_ref[...],
                    0.0)
    o_ref[...] = dinv * jnp.dot(h, w_ref[...],
                                preferred_element_type=jnp.float32)


_mid = pl.pallas_call(
    _mid_body,
    grid=(G,),
    in_specs=[
        pl.BlockSpec((NC, BR, D), lambda i: (0, i, 0)),
        pl.BlockSpec((NC, BR, D), lambda i: (0, i, 0)),
        pl.BlockSpec((BR, D), lambda i: (i, 0)),
        pl.BlockSpec((1, D), lambda i: (0, 0)),
        pl.BlockSpec((D, D), lambda i: (0, 0)),
    ],
    out_specs=pl.BlockSpec((BR, D), lambda i: (i, 0)),
    out_shape=jax.ShapeDtypeStruct((NP, D), jnp.float32),
)


def _fin_body(dp_ref, a_ref, y_ref, b_ref, w_ref, blin_ref, o_ref):
    dinv = lax.rsqrt(1.0 + dp_ref[0] + dp_ref[1])
    h = jnp.maximum(dinv * (a_ref[0] + a_ref[1] + y_ref[...]) + b_ref[...],
                    0.0)
    o_ref[...] = jnp.dot(h, w_ref[...],
                         preferred_element_type=jnp.float32) + blin_ref[...]


_fin = pl.pallas_call(
    _fin_body,
    grid=(G,),
    in_specs=[
        pl.BlockSpec((NC, BR, D), lambda i: (0, i, 0)),
        pl.BlockSpec((NC, BR, D), lambda i: (0, i, 0)),
        pl.BlockSpec((BR, D), lambda i: (i, 0)),
        pl.BlockSpec((1, D), lambda i: (0, 0)),
        pl.BlockSpec((D, NCLS), lambda i: (0, 0)),
        pl.BlockSpec((1, NCLS), lambda i: (0, 0)),
    ],
    out_specs=pl.BlockSpec((BR, NCLS), lambda i: (i, 0)),
    out_shape=jax.ShapeDtypeStruct((NP, NCLS), jnp.float32),
)


def kernel(x, edge_index, W1, b1, W2, b2, Wlin, blin):
    src = edge_index[0].astype(jnp.int32)
    dst = edge_index[1].astype(jnp.int32)
    pad_z = N + (jnp.arange(EPAD - NE, dtype=jnp.int32) % NZ)
    src_p = jnp.concatenate([src, pad_z]).reshape(NW, NB, EB)
    dst_p = jnp.concatenate([dst, pad_z]).reshape(NW, NB, EB)
    x_p = jnp.zeros((NP, D), jnp.float32).at[:N].set(x)
    zer_np = jnp.zeros((NP, D), jnp.float32)
    ones_b = jnp.ones((EB, D), jnp.float32)

    dp = _deg(dst_p, zer_np, ones_b)
    y1 = _lin1(dp, x_p, W1)
    a1 = _agg(y1, src_p, dst_p, zer_np)
    y2 = _mid(dp, a1, y1, b1.reshape(1, D), W2)
    a2 = _agg(y2, src_p, dst_p, zer_np)
    out = _fin(dp, a2, y2, b2.reshape(1, D), Wlin,
               blin.reshape(1, NCLS))
    return out[:N]


# speedup vs baseline: 1.2003x; 1.2003x over previous
import jax
import jax.numpy as jnp
from jax import lax
from jax.experimental import pallas as pl
from jax.experimental.pallas import tpu as pltpu
from jax.experimental.pallas import tpu_sc as plsc

N = 10000
D = 128
NCLS = 64
NE = 320000
NP = 10240
NC = 2
NS = 16
NW = NC * NS
L = 16
EB = 128
NB = 80
NBH = NB // 2
EPW = NB * EB
EPAD = NW * EPW
RPW = NP // NS
HB = 64
HPS = NP // NC // NS
NZ = NP - N

_mesh = plsc.VectorSubcoreMesh(core_axis_name="c", subcore_axis_name="s",
                               num_cores=NC, num_subcores=NS)


def _deg_body(dst_hbm, zer_hbm, ones_hbm, out_hbm, dst_v, ones_v, acc_sh):
    c = lax.axis_index("c")
    s = lax.axis_index("s")
    wid = s * NC + c
    pltpu.sync_copy(dst_hbm.at[wid], dst_v)
    pltpu.sync_copy(ones_hbm, ones_v)
    pltpu.sync_copy(zer_hbm.at[pl.ds(s * RPW, RPW)],
                    acc_sh.at[pl.ds(s * RPW, RPW)])
    plsc.subcore_barrier()

    def step(j, carry):
        pltpu.sync_copy(ones_v, acc_sh.at[dst_v.at[j]], add=True)
        return carry

    lax.fori_loop(0, NB, step, 0)
    plsc.subcore_barrier()
    pltpu.sync_copy(acc_sh.at[pl.ds(s * RPW, RPW)],
                    out_hbm.at[c, pl.ds(s * RPW, RPW)])


_deg = pl.kernel(
    _deg_body,
    out_type=jax.ShapeDtypeStruct((NC, NP, D), jnp.float32),
    mesh=_mesh,
    scratch_types=[
        pltpu.VMEM((NB, EB), jnp.int32),
        pltpu.VMEM((EB, D), jnp.float32),
        pltpu.VMEM_SHARED((NP, D), jnp.float32),
    ],
)


def _agg_body(y_hbm, src_hbm, dst_hbm, zer_hbm, out_hbm,
              src_v, dst_v, rows_v, acc_sh, sem0, sem1):
    c = lax.axis_index("c")
    s = lax.axis_index("s")
    wid = s * NC + c
    pltpu.sync_copy(zer_hbm.at[pl.ds(s * RPW, RPW)],
                    acc_sh.at[pl.ds(s * RPW, RPW)])
    plsc.subcore_barrier()

    for ph in range(2):
        base = ph * NBH
        pltpu.sync_copy(src_hbm.at[wid, pl.ds(base, NBH)], src_v)
        pltpu.sync_copy(dst_hbm.at[wid, pl.ds(base, NBH)], dst_v)
        pltpu.async_copy(y_hbm.at[src_v.at[0]], rows_v.at[0], sem0)
        pltpu.async_copy(y_hbm.at[src_v.at[1]], rows_v.at[1], sem1)

        def step(t, carry):
            j0 = 2 * t
            pltpu.make_async_copy(y_hbm.at[src_v.at[j0]],
                                  rows_v.at[0], sem0).wait()
            pltpu.sync_copy(rows_v.at[0], acc_sh.at[dst_v.at[j0]], add=True)
            pltpu.async_copy(y_hbm.at[src_v.at[j0 + 2]], rows_v.at[0], sem0)
            pltpu.make_async_copy(y_hbm.at[src_v.at[j0 + 1]],
                                  rows_v.at[1], sem1).wait()
            pltpu.sync_copy(rows_v.at[1], acc_sh.at[dst_v.at[j0 + 1]],
                            add=True)
            pltpu.async_copy(y_hbm.at[src_v.at[j0 + 3]], rows_v.at[1], sem1)
            return carry

        lax.fori_loop(0, NBH // 2 - 1, step, 0)
        pltpu.make_async_copy(y_hbm.at[src_v.at[NBH - 2]],
                              rows_v.at[0], sem0).wait()
        pltpu.sync_copy(rows_v.at[0], acc_sh.at[dst_v.at[NBH - 2]], add=True)
        pltpu.make_async_copy(y_hbm.at[src_v.at[NBH - 1]],
                              rows_v.at[1], sem1).wait()
        pltpu.sync_copy(rows_v.at[1], acc_sh.at[dst_v.at[NBH - 1]], add=True)

    plsc.subcore_barrier()
    pltpu.sync_copy(acc_sh.at[pl.ds(s * RPW, RPW)],
                    out_hbm.at[c, pl.ds(s * RPW, RPW)])


_agg = pl.kernel(
    _agg_body,
    out_type=jax.ShapeDtypeStruct((NC, NP, D), jnp.float32),
    mesh=_mesh,
    scratch_types=[
        pltpu.VMEM((NBH, EB), jnp.int32),
        pltpu.VMEM((NBH, EB), jnp.int32),
        pltpu.VMEM((2, EB, D), jnp.float32),
        pltpu.VMEM_SHARED((NP, D), jnp.float32),
        pltpu.SemaphoreType.DMA,
        pltpu.SemaphoreType.DMA,
    ],
)


BR = 1024
G = NP // BR


def _lin1_body(dp_ref, x_ref, w_ref, o_ref):
    dinv = lax.rsqrt(1.0 + dp_ref[0] + dp_ref[1])
    o_ref[...] = dinv * jnp.dot(x_ref[...], w_ref[...],
                                preferred_element_type=jnp.float32)


_lin1 = pl.pallas_call(
    _lin1_body,
    grid=(G,),
    in_specs=[
        pl.BlockSpec((NC, BR, D), lambda i: (0, i, 0)),
        pl.BlockSpec((BR, D), lambda i: (i, 0)),
        pl.BlockSpec((D, D), lambda i: (0, 0)),
    ],
    out_specs=pl.BlockSpec((BR, D), lambda i: (i, 0)),
    out_shape=jax.ShapeDtypeStruct((NP, D), jnp.float32),
)


def _mid_body(dp_ref, a_ref, y_ref, b_ref, w_ref, o_ref):
    dinv = lax.rsqrt(1.0 + dp_ref[0] + dp_ref[1])
    h = jnp.maximum(dinv * (a_ref[0] + a_ref[1] + y_ref[...]) + b_ref[...],
                    0.0)
    o_ref[...] = dinv * jnp.dot(h, w_ref[...],
                                preferred_element_type=jnp.float32)


_mid = pl.pallas_call(
    _mid_body,
    grid=(G,),
    in_specs=[
        pl.BlockSpec((NC, BR, D), lambda i: (0, i, 0)),
        pl.BlockSpec((NC, BR, D), lambda i: (0, i, 0)),
        pl.BlockSpec((BR, D), lambda i: (i, 0)),
        pl.BlockSpec((1, D), lambda i: (0, 0)),
        pl.BlockSpec((D, D), lambda i: (0, 0)),
    ],
    out_specs=pl.BlockSpec((BR, D), lambda i: (i, 0)),
    out_shape=jax.ShapeDtypeStruct((NP, D), jnp.float32),
)


def _fin_body(dp_ref, a_ref, y_ref, b_ref, w_ref, blin_ref, o_ref):
    dinv = lax.rsqrt(1.0 + dp_ref[0] + dp_ref[1])
    h = jnp.maximum(dinv * (a_ref[0] + a_ref[1] + y_ref[...]) + b_ref[...],
                    0.0)
    o_ref[...] = jnp.dot(h, w_ref[...],
                         preferred_element_type=jnp.float32) + blin_ref[...]


_fin = pl.pallas_call(
    _fin_body,
    grid=(G,),
    in_specs=[
        pl.BlockSpec((NC, BR, D), lambda i: (0, i, 0)),
        pl.BlockSpec((NC, BR, D), lambda i: (0, i, 0)),
        pl.BlockSpec((BR, D), lambda i: (i, 0)),
        pl.BlockSpec((1, D), lambda i: (0, 0)),
        pl.BlockSpec((D, NCLS), lambda i: (0, 0)),
        pl.BlockSpec((1, NCLS), lambda i: (0, 0)),
    ],
    out_specs=pl.BlockSpec((BR, NCLS), lambda i: (i, 0)),
    out_shape=jax.ShapeDtypeStruct((NP, NCLS), jnp.float32),
)


def kernel(x, edge_index, W1, b1, W2, b2, Wlin, blin):
    src = edge_index[0].astype(jnp.int32)
    dst = edge_index[1].astype(jnp.int32)
    pad_z = N + (jnp.arange(EPAD - NE, dtype=jnp.int32) % NZ)
    src_p = jnp.concatenate([src, pad_z]).reshape(NW, NB, EB)
    dst_p = jnp.concatenate([dst, pad_z]).reshape(NW, NB, EB)
    x_p = jnp.zeros((NP, D), jnp.float32).at[:N].set(x)
    zer_np = jnp.zeros((NP, D), jnp.float32)
    ones_b = jnp.ones((EB, D), jnp.float32)

    dp = _deg(dst_p, zer_np, ones_b)
    y1 = _lin1(dp, x_p, W1)
    a1 = _agg(y1, src_p, dst_p, zer_np)
    y2 = _mid(dp, a1, y1, b1.reshape(1, D), W2)
    a2 = _agg(y2, src_p, dst_p, zer_np)
    out = _fin(dp, a2, y2, b2.reshape(1, D), Wlin,
               blin.reshape(1, NCLS))
    return out[:N]
